# Initial kernel scaffold; baseline (speedup 1.0000x reference)
import jax
import jax.numpy as jnp
from jax.experimental import pallas as pl


def kernel(current_state, edge_index, efeatures, params):
    raise NotImplementedError("write your pallas kernel here")


# speedup vs baseline: 13.5436x; 13.5436x over previous
import functools

import jax
import jax.numpy as jnp
from jax import lax
from jax.experimental import pallas as pl
from jax.experimental.pallas import tpu as pltpu
from jax.experimental.pallas import tpu_sc as plsc

_N = 50000
_E = 800000
_LN_EPS = 1e-5

_F32 = jnp.float32


def _lrelu(x):
    return jnp.where(x > 0, x, x * 0.01)


def _kron8(w):
    return jnp.kron(jnp.eye(8, dtype=w.dtype), w)


def _t8(v):
    return jnp.tile(v, 8)[None, :]


def _mean_mat():
    return jnp.kron(jnp.eye(8, dtype=_F32), jnp.full((16, 16), 1.0 / 16.0, _F32))


def _full_spec(a):
    nd = a.ndim
    return pl.BlockSpec(a.shape, lambda i, _nd=nd: (0,) * _nd)


def _packed_ln_weights(p):
    mm = _mean_mat()
    ko = _kron8(p["out"]["W"])
    bo = _t8(p["out"]["b"])
    return dict(
        kh=_kron8(p["hidden"][0]["W"]),
        kom=jnp.concatenate([ko, ko @ mm], axis=1),
        bo2=jnp.concatenate([bo, bo @ mm], axis=1),
        bh=_t8(p["hidden"][0]["b"]),
        g=_t8(p["ln"]["g"]),
        bln=_t8(p["ln"]["b"]),
        mm=mm,
    )


def _mlp_tail_ln(h1, w):
    h2 = _lrelu(jnp.dot(h1, w["kh"], preferred_element_type=_F32) + w["bh"])
    fm = jnp.dot(h2, w["kom"], preferred_element_type=_F32) + w["bo2"]
    f = fm[:, :128]
    mu = fm[:, 128:]
    d = f - mu
    var = jnp.dot(d * d, w["mm"], preferred_element_type=_F32)
    return d * lax.rsqrt(var + _LN_EPS) * w["g"] + w["bln"]


def _edge_proc(pe_p, ga_p, gb_p, p):
    e8 = pe_p.shape[0]
    be = 2000
    w = _packed_ln_weights(p)
    k0 = _kron8(p["inp"]["W"][0:16])
    k1 = _kron8(p["inp"]["W"][16:32])
    k2 = _kron8(p["inp"]["W"][32:48])
    b1 = _t8(p["inp"]["b"])

    def body(pe, ga, gb, k0r, k1r, k2r, khr, komr, mmr, b1r, bhr, bo2r, gr,
             blnr, o):
        x = pe[...]
        pre = (jnp.dot(x, k0r[...], preferred_element_type=_F32)
               + jnp.dot(ga[...], k1r[...], preferred_element_type=_F32)
               + jnp.dot(gb[...], k2r[...], preferred_element_type=_F32)
               + b1r[...])
        wd = dict(kh=khr[...], kom=komr[...], mm=mmr[...], bh=bhr[...],
                  bo2=bo2r[...], g=gr[...], bln=blnr[...])
        o[...] = _mlp_tail_ln(_lrelu(pre), wd) + x

    blk = pl.BlockSpec((be, 128), lambda i: (i, 0))
    ws = [k0, k1, k2, w["kh"], w["kom"], w["mm"], b1, w["bh"], w["bo2"],
          w["g"], w["bln"]]
    return pl.pallas_call(
        body,
        grid=(e8 // be,),
        in_specs=[blk, blk, blk] + [_full_spec(a) for a in ws],
        out_specs=blk,
        out_shape=jax.ShapeDtypeStruct((e8, 128), _F32),
    )(pe_p, ga_p, gb_p, *ws)


def _node_proc(pn_p, agg2_p, p):
    n8 = pn_p.shape[0]
    bn = n8
    w = _packed_ln_weights(p)
    k0 = _kron8(p["inp"]["W"][0:16])
    k1 = _kron8(p["inp"]["W"][16:32])
    b1 = _t8(p["inp"]["b"])

    def body(pn, ag, k0r, k1r, khr, komr, mmr, b1r, bhr, bo2r, gr, blnr, o):
        x = pn[...]
        agg = ag[0] + ag[1]
        pre = (jnp.dot(x, k0r[...], preferred_element_type=_F32)
               + jnp.dot(agg, k1r[...], preferred_element_type=_F32)
               + b1r[...])
        wd = dict(kh=khr[...], kom=komr[...], mm=mmr[...], bh=bhr[...],
                  bo2=bo2r[...], g=gr[...], bln=blnr[...])
        o[...] = _mlp_tail_ln(_lrelu(pre), wd) + x

    blk = pl.BlockSpec((bn, 128), lambda i: (i, 0))
    blk2 = pl.BlockSpec((2, bn, 128), lambda i: (0, i, 0))
    ws = [k0, k1, w["kh"], w["kom"], w["mm"], b1, w["bh"], w["bo2"], w["g"],
          w["bln"]]
    return pl.pallas_call(
        body,
        grid=(n8 // bn,),
        in_specs=[blk, blk2] + [_full_spec(a) for a in ws],
        out_specs=blk,
        out_shape=jax.ShapeDtypeStruct((n8, 128), _F32),
    )(pn_p, agg2_p, *ws)


def _node_encoder(x, p):
    n = x.shape[0]
    bn = 2000

    def body(xr, wir, bir, whr, bhr, wor, bor, gr, blnr, o):
        h = _lrelu(jnp.dot(xr[...], wir[...], preferred_element_type=_F32)
                   + bir[...])
        h = _lrelu(jnp.dot(h, whr[...], preferred_element_type=_F32)
                   + bhr[...])
        f = jnp.dot(h, wor[...], preferred_element_type=_F32) + bor[...]
        mu = jnp.mean(f, axis=-1, keepdims=True)
        d = f - mu
        var = jnp.mean(d * d, axis=-1, keepdims=True)
        o[...] = d * lax.rsqrt(var + _LN_EPS) * gr[...] + blnr[...]

    ws = [p["inp"]["W"], p["inp"]["b"][None, :], p["hidden"][0]["W"],
          p["hidden"][0]["b"][None, :], p["out"]["W"], p["out"]["b"][None, :],
          p["ln"]["g"][None, :], p["ln"]["b"][None, :]]
    return pl.pallas_call(
        body,
        grid=(n // bn,),
        in_specs=[pl.BlockSpec((bn, 128), lambda i: (i, 0))]
        + [_full_spec(a) for a in ws],
        out_specs=pl.BlockSpec((bn, 16), lambda i: (i, 0)),
        out_shape=jax.ShapeDtypeStruct((n, 16), _F32),
    )(x, *ws)


def _edge_encoder(ef_p, p_red, p_rec):
    e8 = ef_p.shape[0]
    be = 2000
    wred = _packed_ln_weights(p_red)
    wrec = _packed_ln_weights(p_rec)
    k0 = jnp.concatenate([_kron8(p_red["inp"]["W"]),
                          _kron8(p_rec["inp"]["W"])], axis=1)
    b1 = jnp.concatenate([_t8(p_red["inp"]["b"]), _t8(p_rec["inp"]["b"])],
                         axis=1)

    def body(ef, k0r, b1r, kh1, kom1, bh1, bo21, g1, bln1, kh2, kom2, bh2,
             bo22, g2, bln2, mmr, o1, o2):
        pre = jnp.dot(ef[...], k0r[...], preferred_element_type=_F32) + b1r[...]
        mm = mmr[...]
        w1 = dict(kh=kh1[...], kom=kom1[...], mm=mm, bh=bh1[...],
                  bo2=bo21[...], g=g1[...], bln=bln1[...])
        w2 = dict(kh=kh2[...], kom=kom2[...], mm=mm, bh=bh2[...],
                  bo2=bo22[...], g=g2[...], bln=bln2[...])
        o1[...] = _mlp_tail_ln(_lrelu(pre[:, :128]), w1)
        o2[...] = _mlp_tail_ln(_lrelu(pre[:, 128:]), w2)

    blk = pl.BlockSpec((be, 128), lambda i: (i, 0))
    ws = [k0, b1,
          wred["kh"], wred["kom"], wred["bh"], wred["bo2"], wred["g"],
          wred["bln"],
          wrec["kh"], wrec["kom"], wrec["bh"], wrec["bo2"], wrec["g"],
          wrec["bln"], wred["mm"]]
    return pl.pallas_call(
        body,
        grid=(e8 // be,),
        in_specs=[blk] + [_full_spec(a) for a in ws],
        out_specs=[blk, blk],
        out_shape=[jax.ShapeDtypeStruct((e8, 128), _F32),
                   jax.ShapeDtypeStruct((e8, 128), _F32)],
    )(ef_p, *ws)


def _mid_decoder_encoder(pn_p, p_dec, p_enc):
    n8 = pn_p.shape[0]
    bn = n8
    kd0 = _kron8(p_dec["inp"]["W"])
    kdh = _kron8(p_dec["hidden"][0]["W"])
    kdo = _kron8(p_dec["out"]["W"])
    bd0 = _t8(p_dec["inp"]["b"])
    bdh = _t8(p_dec["hidden"][0]["b"])
    bdo = _t8(p_dec["out"]["b"])
    we = _packed_ln_weights(p_enc)
    ke0 = _kron8(p_enc["inp"]["W"])
    be0 = _t8(p_enc["inp"]["b"])

    def body(pn, kd0r, kdhr, kdor, bd0r, bdhr, bdor, ke0r, be0r, khr, komr,
             mmr, bhr, bo2r, gr, blnr, o):
        x = pn[...]
        h = _lrelu(jnp.dot(x, kd0r[...], preferred_element_type=_F32)
                   + bd0r[...])
        h = _lrelu(jnp.dot(h, kdhr[...], preferred_element_type=_F32)
                   + bdhr[...])
        hlat = jnp.dot(h, kdor[...], preferred_element_type=_F32) + bdor[...]
        pre = (jnp.dot(hlat, ke0r[...], preferred_element_type=_F32)
               + be0r[...])
        wd = dict(kh=khr[...], kom=komr[...], mm=mmr[...], bh=bhr[...],
                  bo2=bo2r[...], g=gr[...], bln=blnr[...])
        o[...] = _mlp_tail_ln(_lrelu(pre), wd)

    blk = pl.BlockSpec((bn, 128), lambda i: (i, 0))
    ws = [kd0, kdh, kdo, bd0, bdh, bdo, ke0, be0, we["kh"], we["kom"],
          we["mm"], we["bh"], we["bo2"], we["g"], we["bln"]]
    return pl.pallas_call(
        body,
        grid=(n8 // bn,),
        in_specs=[blk] + [_full_spec(a) for a in ws],
        out_specs=blk,
        out_shape=jax.ShapeDtypeStruct((n8, 128), _F32),
    )(pn_p, *ws)


def _final_decoder(pn_p, p):
    n8 = pn_p.shape[0]
    bn = n8
    k0 = _kron8(p["inp"]["W"])
    kh = _kron8(p["hidden"][0]["W"])
    ko = _kron8(p["out"]["W"])
    b0 = _t8(p["inp"]["b"])
    bh = _t8(p["hidden"][0]["b"])
    bo = _t8(p["out"]["b"])

    def body(pn, k0r, khr, kor, b0r, bhr, bor, o):
        h = _lrelu(jnp.dot(pn[...], k0r[...], preferred_element_type=_F32)
                   + b0r[...])
        h = _lrelu(jnp.dot(h, khr[...], preferred_element_type=_F32)
                   + bhr[...])
        o[...] = jnp.dot(h, kor[...], preferred_element_type=_F32) + bor[...]

    ws = [k0, kh, ko, b0, bh, bo]
    return pl.pallas_call(
        body,
        grid=(n8 // bn,),
        in_specs=[pl.BlockSpec((bn, 128), lambda i: (i, 0))]
        + [_full_spec(a) for a in ws],
        out_specs=pl.BlockSpec((bn, 16), lambda i: (i, 0)),
        out_shape=jax.ShapeDtypeStruct((n8, 16), _F32),
    )(pn_p, *ws)


@functools.cache
def _sc_mesh():
    return plsc.VectorSubcoreMesh(core_axis_name="c", subcore_axis_name="s")


_NTILES = 32
_EPT = _E // _NTILES
_GCH = 1000
_NCH = _EPT // _GCH


def _sc_gather(pn, src, dst):
    def body(pn_h, src_h, dst_h, gs_h, gd_h, idx_s, idx_d, buf_s, buf_d,
             sem_g, sem_w):
        wid = lax.axis_index("s") * 2 + lax.axis_index("c")
        base = pl.multiple_of(wid * _EPT, 8)
        pltpu.sync_copy(src_h.at[pl.ds(base, _EPT)], idx_s)
        pltpu.sync_copy(dst_h.at[pl.ds(base, _EPT)], idx_d)
        writes = []
        for i in range(_NCH):
            b = (i % 2) * _GCH
            if i >= 2:
                for d in writes[i - 2]:
                    d.wait()
            ca = pltpu.async_copy(pn_h.at[idx_s.at[pl.ds(i * _GCH, _GCH)]],
                                  buf_s.at[pl.ds(b, _GCH)], sem_g)
            cb = pltpu.async_copy(pn_h.at[idx_d.at[pl.ds(i * _GCH, _GCH)]],
                                  buf_d.at[pl.ds(b, _GCH)], sem_g)
            ca.wait()
            cb.wait()
            w1 = pltpu.async_copy(buf_s.at[pl.ds(b, _GCH)],
                                  gs_h.at[pl.ds(base + i * _GCH, _GCH)], sem_w)
            w2 = pltpu.async_copy(buf_d.at[pl.ds(b, _GCH)],
                                  gd_h.at[pl.ds(base + i * _GCH, _GCH)], sem_w)
            writes.append((w1, w2))
        for pair in writes[-2:]:
            for d in pair:
                d.wait()

    f = pl.kernel(
        body,
        out_type=(jax.ShapeDtypeStruct((_E, 16), _F32),
                  jax.ShapeDtypeStruct((_E, 16), _F32)),
        mesh=_sc_mesh(),
        compiler_params=pltpu.CompilerParams(use_tc_tiling_on_sc=False),
        scratch_types=[
            pltpu.VMEM((_EPT,), jnp.int32),
            pltpu.VMEM((_EPT,), jnp.int32),
            pltpu.VMEM((2 * _GCH, 16), _F32),
            pltpu.VMEM((2 * _GCH, 16), _F32),
            pltpu.SemaphoreType.DMA,
            pltpu.SemaphoreType.DMA,
        ],
    )
    return f(pn, src, dst)


_SRPD = 125
_SROW = _E // _SRPD
_SRPT = _SROW // _NTILES
_NPT = _N // 16


def _sc_scatter(pe, dst2, zeros):
    def body(pe_h, dst_h, z_h, agg_h, idx2, buf, table, sem_s, sem_w):
        cid = lax.axis_index("c")
        sid = lax.axis_index("s")
        wid = sid * 2 + cid
        pltpu.sync_copy(z_h.at[pl.ds(sid * _NPT, _NPT)],
                        table.at[pl.ds(sid * _NPT, _NPT)])
        plsc.subcore_barrier()
        pltpu.sync_copy(dst_h.at[pl.ds(wid * _SRPT, _SRPT)], idx2)
        for i in range(_NCH):
            pltpu.sync_copy(pe_h.at[pl.ds(wid * _EPT + i * _GCH, _GCH)], buf)
            descs = []
            for g in range(8):
                d = pltpu.async_copy(buf.at[pl.ds(g * _SRPD, _SRPD)],
                                     table.at[idx2.at[i * 8 + g]], sem_s,
                                     add=True)
                descs.append(d)
            for d in descs:
                d.wait()
        plsc.subcore_barrier()
        pltpu.sync_copy(table.at[pl.ds(sid * _NPT, _NPT)],
                        agg_h.at[cid, pl.ds(sid * _NPT, _NPT)])

    f = pl.kernel(
        body,
        out_type=jax.ShapeDtypeStruct((2, _N, 16), _F32),
        mesh=_sc_mesh(),
        compiler_params=pltpu.CompilerParams(use_tc_tiling_on_sc=False),
        scratch_types=[
            pltpu.VMEM((_SRPT, _SRPD), jnp.int32),
            pltpu.VMEM((_GCH, 16), _F32),
            pltpu.VMEM_SHARED((_N, 16), _F32),
            pltpu.SemaphoreType.DMA,
            pltpu.SemaphoreType.DMA,
        ],
    )
    return f(pe, dst2, zeros)


def kernel(current_state, edge_index, efeatures, params):
    src = edge_index[0].astype(jnp.int32)
    dst = edge_index[1].astype(jnp.int32)
    dst2 = dst.reshape(_SROW, _SRPD)
    zeros = jnp.zeros((_N, 16), _F32)
    n8 = _N // 8

    ef_p = efeatures.reshape(_E // 8, 128)
    pe_red_p, pe_rec_p = _edge_encoder(ef_p, params["enc_e_red"],
                                       params["enc_e_rec"])
    pn = _node_encoder(current_state, params["enc_n_red"])
    pn_p = pn.reshape(n8, 128)

    def mp_pass(pn_p, pe_p, e_params, n_params):
        for i in range(3):
            gs, gd = _sc_gather(pn_p.reshape(_N, 16), src, dst)
            pe_p = _edge_proc(pe_p, gs.reshape(_E // 8, 128),
                              gd.reshape(_E // 8, 128), e_params[i])
            agg2 = _sc_scatter(pe_p.reshape(_E, 16), dst2, zeros)
            pn_p = _node_proc(pn_p, agg2.reshape(2, n8, 128), n_params[i])
        return pn_p

    pn_p = mp_pass(pn_p, pe_red_p, params["proc_e_red"], params["proc_n_red"])
    pn_p = _mid_decoder_encoder(pn_p, params["dec_n_red"], params["enc_n_rec"])
    pn_p = mp_pass(pn_p, pe_rec_p, params["proc_e_rec"], params["proc_n_rec"])
    out_p = _final_decoder(pn_p, params["dec_n_rec"])
    return out_p.reshape(_N, 2)


# speedup vs baseline: 14.0543x; 1.0377x over previous
import functools

import jax
import jax.numpy as jnp
from jax import lax
from jax.experimental import pallas as pl
from jax.experimental.pallas import tpu as pltpu
from jax.experimental.pallas import tpu_sc as plsc

_N = 50000
_E = 800000
_LN_EPS = 1e-5

_F32 = jnp.float32


def _lrelu(x):
    return jnp.where(x > 0, x, x * 0.01)


def _kron8(w):
    return jnp.kron(jnp.eye(8, dtype=w.dtype), w)


def _t8(v):
    return jnp.tile(v, 8)[None, :]


def _mean_mat():
    return jnp.kron(jnp.eye(8, dtype=_F32), jnp.full((16, 16), 1.0 / 16.0, _F32))


def _full_spec(a):
    nd = a.ndim
    return pl.BlockSpec(a.shape, lambda i, _nd=nd: (0,) * _nd)


def _packed_ln_weights(p):
    mm = _mean_mat()
    ko = _kron8(p["out"]["W"])
    bo = _t8(p["out"]["b"])
    return dict(
        kh=_kron8(p["hidden"][0]["W"]),
        kom=jnp.concatenate([ko, ko @ mm], axis=1),
        bo2=jnp.concatenate([bo, bo @ mm], axis=1),
        bh=_t8(p["hidden"][0]["b"]),
        g=_t8(p["ln"]["g"]),
        bln=_t8(p["ln"]["b"]),
        mm=mm,
    )


def _mlp_tail_ln(h1, w):
    h2 = _lrelu(jnp.dot(h1, w["kh"], preferred_element_type=_F32) + w["bh"])
    fm = jnp.dot(h2, w["kom"], preferred_element_type=_F32) + w["bo2"]
    f = fm[:, :128]
    mu = fm[:, 128:]
    d = f - mu
    var = jnp.dot(d * d, w["mm"], preferred_element_type=_F32)
    return d * lax.rsqrt(var + _LN_EPS) * w["g"] + w["bln"]


def _edge_proc(pe_p, ga_p, gb_p, p):
    e8 = pe_p.shape[0]
    be = 2000
    w = _packed_ln_weights(p)
    k0 = _kron8(p["inp"]["W"][0:16])
    k1 = _kron8(p["inp"]["W"][16:32])
    k2 = _kron8(p["inp"]["W"][32:48])
    b1 = _t8(p["inp"]["b"])

    def body(pe, ga, gb, k0r, k1r, k2r, khr, komr, mmr, b1r, bhr, bo2r, gr,
             blnr, o):
        x = pe[...]
        pre = (jnp.dot(x, k0r[...], preferred_element_type=_F32)
               + jnp.dot(ga[...], k1r[...], preferred_element_type=_F32)
               + jnp.dot(gb[...], k2r[...], preferred_element_type=_F32)
               + b1r[...])
        wd = dict(kh=khr[...], kom=komr[...], mm=mmr[...], bh=bhr[...],
                  bo2=bo2r[...], g=gr[...], bln=blnr[...])
        o[...] = _mlp_tail_ln(_lrelu(pre), wd) + x

    blk = pl.BlockSpec((be, 128), lambda i: (i, 0))
    ws = [k0, k1, k2, w["kh"], w["kom"], w["mm"], b1, w["bh"], w["bo2"],
          w["g"], w["bln"]]
    return pl.pallas_call(
        body,
        grid=(e8 // be,),
        in_specs=[blk, blk, blk] + [_full_spec(a) for a in ws],
        out_specs=blk,
        out_shape=jax.ShapeDtypeStruct((e8, 128), _F32),
    )(pe_p, ga_p, gb_p, *ws)


def _node_proc(pn_p, agg2_p, p):
    n8 = pn_p.shape[0]
    bn = n8
    w = _packed_ln_weights(p)
    k0 = _kron8(p["inp"]["W"][0:16])
    k1 = _kron8(p["inp"]["W"][16:32])
    b1 = _t8(p["inp"]["b"])

    def body(pn, ag, k0r, k1r, khr, komr, mmr, b1r, bhr, bo2r, gr, blnr, o):
        x = pn[...]
        agg = ag[0] + ag[1]
        pre = (jnp.dot(x, k0r[...], preferred_element_type=_F32)
               + jnp.dot(agg, k1r[...], preferred_element_type=_F32)
               + b1r[...])
        wd = dict(kh=khr[...], kom=komr[...], mm=mmr[...], bh=bhr[...],
                  bo2=bo2r[...], g=gr[...], bln=blnr[...])
        o[...] = _mlp_tail_ln(_lrelu(pre), wd) + x

    blk = pl.BlockSpec((bn, 128), lambda i: (i, 0))
    blk2 = pl.BlockSpec((2, bn, 128), lambda i: (0, i, 0))
    ws = [k0, k1, w["kh"], w["kom"], w["mm"], b1, w["bh"], w["bo2"], w["g"],
          w["bln"]]
    return pl.pallas_call(
        body,
        grid=(n8 // bn,),
        in_specs=[blk, blk2] + [_full_spec(a) for a in ws],
        out_specs=blk,
        out_shape=jax.ShapeDtypeStruct((n8, 128), _F32),
    )(pn_p, agg2_p, *ws)


def _node_encoder(x, p):
    n = x.shape[0]
    bn = 2000

    def body(xr, wir, bir, whr, bhr, wor, bor, gr, blnr, o):
        h = _lrelu(jnp.dot(xr[...], wir[...], preferred_element_type=_F32)
                   + bir[...])
        h = _lrelu(jnp.dot(h, whr[...], preferred_element_type=_F32)
                   + bhr[...])
        f = jnp.dot(h, wor[...], preferred_element_type=_F32) + bor[...]
        mu = jnp.mean(f, axis=-1, keepdims=True)
        d = f - mu
        var = jnp.mean(d * d, axis=-1, keepdims=True)
        o[...] = d * lax.rsqrt(var + _LN_EPS) * gr[...] + blnr[...]

    ws = [p["inp"]["W"], p["inp"]["b"][None, :], p["hidden"][0]["W"],
          p["hidden"][0]["b"][None, :], p["out"]["W"], p["out"]["b"][None, :],
          p["ln"]["g"][None, :], p["ln"]["b"][None, :]]
    return pl.pallas_call(
        body,
        grid=(n // bn,),
        in_specs=[pl.BlockSpec((bn, 128), lambda i: (i, 0))]
        + [_full_spec(a) for a in ws],
        out_specs=pl.BlockSpec((bn, 16), lambda i: (i, 0)),
        out_shape=jax.ShapeDtypeStruct((n, 16), _F32),
    )(x, *ws)


def _edge_encoder(ef_p, p_red, p_rec):
    e8 = ef_p.shape[0]
    be = 2000
    wred = _packed_ln_weights(p_red)
    wrec = _packed_ln_weights(p_rec)
    k0 = jnp.concatenate([_kron8(p_red["inp"]["W"]),
                          _kron8(p_rec["inp"]["W"])], axis=1)
    b1 = jnp.concatenate([_t8(p_red["inp"]["b"]), _t8(p_rec["inp"]["b"])],
                         axis=1)

    def body(ef, k0r, b1r, kh1, kom1, bh1, bo21, g1, bln1, kh2, kom2, bh2,
             bo22, g2, bln2, mmr, o1, o2):
        pre = jnp.dot(ef[...], k0r[...], preferred_element_type=_F32) + b1r[...]
        mm = mmr[...]
        w1 = dict(kh=kh1[...], kom=kom1[...], mm=mm, bh=bh1[...],
                  bo2=bo21[...], g=g1[...], bln=bln1[...])
        w2 = dict(kh=kh2[...], kom=kom2[...], mm=mm, bh=bh2[...],
                  bo2=bo22[...], g=g2[...], bln=bln2[...])
        o1[...] = _mlp_tail_ln(_lrelu(pre[:, :128]), w1)
        o2[...] = _mlp_tail_ln(_lrelu(pre[:, 128:]), w2)

    blk = pl.BlockSpec((be, 128), lambda i: (i, 0))
    ws = [k0, b1,
          wred["kh"], wred["kom"], wred["bh"], wred["bo2"], wred["g"],
          wred["bln"],
          wrec["kh"], wrec["kom"], wrec["bh"], wrec["bo2"], wrec["g"],
          wrec["bln"], wred["mm"]]
    return pl.pallas_call(
        body,
        grid=(e8 // be,),
        in_specs=[blk] + [_full_spec(a) for a in ws],
        out_specs=[blk, blk],
        out_shape=[jax.ShapeDtypeStruct((e8, 128), _F32),
                   jax.ShapeDtypeStruct((e8, 128), _F32)],
    )(ef_p, *ws)


def _mid_decoder_encoder(pn_p, p_dec, p_enc):
    n8 = pn_p.shape[0]
    bn = n8
    kd0 = _kron8(p_dec["inp"]["W"])
    kdh = _kron8(p_dec["hidden"][0]["W"])
    kdo = _kron8(p_dec["out"]["W"])
    bd0 = _t8(p_dec["inp"]["b"])
    bdh = _t8(p_dec["hidden"][0]["b"])
    bdo = _t8(p_dec["out"]["b"])
    we = _packed_ln_weights(p_enc)
    ke0 = _kron8(p_enc["inp"]["W"])
    be0 = _t8(p_enc["inp"]["b"])

    def body(pn, kd0r, kdhr, kdor, bd0r, bdhr, bdor, ke0r, be0r, khr, komr,
             mmr, bhr, bo2r, gr, blnr, o):
        x = pn[...]
        h = _lrelu(jnp.dot(x, kd0r[...], preferred_element_type=_F32)
                   + bd0r[...])
        h = _lrelu(jnp.dot(h, kdhr[...], preferred_element_type=_F32)
                   + bdhr[...])
        hlat = jnp.dot(h, kdor[...], preferred_element_type=_F32) + bdor[...]
        pre = (jnp.dot(hlat, ke0r[...], preferred_element_type=_F32)
               + be0r[...])
        wd = dict(kh=khr[...], kom=komr[...], mm=mmr[...], bh=bhr[...],
                  bo2=bo2r[...], g=gr[...], bln=blnr[...])
        o[...] = _mlp_tail_ln(_lrelu(pre), wd)

    blk = pl.BlockSpec((bn, 128), lambda i: (i, 0))
    ws = [kd0, kdh, kdo, bd0, bdh, bdo, ke0, be0, we["kh"], we["kom"],
          we["mm"], we["bh"], we["bo2"], we["g"], we["bln"]]
    return pl.pallas_call(
        body,
        grid=(n8 // bn,),
        in_specs=[blk] + [_full_spec(a) for a in ws],
        out_specs=blk,
        out_shape=jax.ShapeDtypeStruct((n8, 128), _F32),
    )(pn_p, *ws)


def _final_decoder(pn_p, p):
    n8 = pn_p.shape[0]
    bn = n8
    k0 = _kron8(p["inp"]["W"])
    kh = _kron8(p["hidden"][0]["W"])
    ko = _kron8(p["out"]["W"])
    b0 = _t8(p["inp"]["b"])
    bh = _t8(p["hidden"][0]["b"])
    bo = _t8(p["out"]["b"])

    def body(pn, k0r, khr, kor, b0r, bhr, bor, o):
        h = _lrelu(jnp.dot(pn[...], k0r[...], preferred_element_type=_F32)
                   + b0r[...])
        h = _lrelu(jnp.dot(h, khr[...], preferred_element_type=_F32)
                   + bhr[...])
        o[...] = jnp.dot(h, kor[...], preferred_element_type=_F32) + bor[...]

    ws = [k0, kh, ko, b0, bh, bo]
    return pl.pallas_call(
        body,
        grid=(n8 // bn,),
        in_specs=[pl.BlockSpec((bn, 128), lambda i: (i, 0))]
        + [_full_spec(a) for a in ws],
        out_specs=pl.BlockSpec((bn, 16), lambda i: (i, 0)),
        out_shape=jax.ShapeDtypeStruct((n8, 16), _F32),
    )(pn_p, *ws)


@functools.cache
def _sc_mesh():
    return plsc.VectorSubcoreMesh(core_axis_name="c", subcore_axis_name="s")


_NTILES = 32
_EPT = _E // _NTILES
_GCH = 1000
_NCH = _EPT // _GCH


def _sc_gather(pn, src, dst):
    def body(pn_h, src_h, dst_h, gs_h, gd_h, idx_s, idx_d, buf_s, buf_d,
             sem_g, sem_w):
        wid = lax.axis_index("s") * 2 + lax.axis_index("c")
        base = pl.multiple_of(wid * _EPT, 8)
        pltpu.sync_copy(src_h.at[pl.ds(base, _EPT)], idx_s)
        pltpu.sync_copy(dst_h.at[pl.ds(base, _EPT)], idx_d)
        writes = []
        for i in range(_NCH):
            b = (i % 2) * _GCH
            if i >= 2:
                for d in writes[i - 2]:
                    d.wait()
            ca = pltpu.async_copy(pn_h.at[idx_s.at[pl.ds(i * _GCH, _GCH)]],
                                  buf_s.at[pl.ds(b, _GCH)], sem_g)
            cb = pltpu.async_copy(pn_h.at[idx_d.at[pl.ds(i * _GCH, _GCH)]],
                                  buf_d.at[pl.ds(b, _GCH)], sem_g)
            ca.wait()
            cb.wait()
            w1 = pltpu.async_copy(buf_s.at[pl.ds(b, _GCH)],
                                  gs_h.at[pl.ds(base + i * _GCH, _GCH)], sem_w)
            w2 = pltpu.async_copy(buf_d.at[pl.ds(b, _GCH)],
                                  gd_h.at[pl.ds(base + i * _GCH, _GCH)], sem_w)
            writes.append((w1, w2))
        for pair in writes[-2:]:
            for d in pair:
                d.wait()

    f = pl.kernel(
        body,
        out_type=(jax.ShapeDtypeStruct((_E, 16), _F32),
                  jax.ShapeDtypeStruct((_E, 16), _F32)),
        mesh=_sc_mesh(),
        compiler_params=pltpu.CompilerParams(use_tc_tiling_on_sc=False),
        scratch_types=[
            pltpu.VMEM((_EPT,), jnp.int32),
            pltpu.VMEM((_EPT,), jnp.int32),
            pltpu.VMEM((2 * _GCH, 16), _F32),
            pltpu.VMEM((2 * _GCH, 16), _F32),
            pltpu.SemaphoreType.DMA,
            pltpu.SemaphoreType.DMA,
        ],
    )
    return f(pn, src, dst)


_SRPD = 125
_SROW = _E // _SRPD
_SRPT = _SROW // _NTILES
_NPT = _N // 16


def _sc_scatter(pe, dst2, zeros):
    def body(pe_h, dst_h, z_h, agg_h, idx2, buf, table, sem_s, sem_l):
        cid = lax.axis_index("c")
        sid = lax.axis_index("s")
        wid = sid * 2 + cid
        pltpu.sync_copy(z_h.at[pl.ds(sid * _NPT, _NPT)],
                        table.at[pl.ds(sid * _NPT, _NPT)])
        plsc.subcore_barrier()
        pltpu.sync_copy(dst_h.at[pl.ds(wid * _SRPT, _SRPT)], idx2)
        loads = [pltpu.async_copy(pe_h.at[pl.ds(wid * _EPT, _GCH)],
                                  buf.at[pl.ds(0, _GCH)], sem_l)]
        for i in range(_NCH):
            b = (i % 2) * _GCH
            loads[i].wait()
            if i + 1 < _NCH:
                nb = ((i + 1) % 2) * _GCH
                loads.append(pltpu.async_copy(
                    pe_h.at[pl.ds(wid * _EPT + (i + 1) * _GCH, _GCH)],
                    buf.at[pl.ds(nb, _GCH)], sem_l))
            descs = []
            for g in range(8):
                d = pltpu.async_copy(buf.at[pl.ds(b + g * _SRPD, _SRPD)],
                                     table.at[idx2.at[i * 8 + g]], sem_s,
                                     add=True)
                descs.append(d)
            for d in descs:
                d.wait()
        plsc.subcore_barrier()
        pltpu.sync_copy(table.at[pl.ds(sid * _NPT, _NPT)],
                        agg_h.at[cid, pl.ds(sid * _NPT, _NPT)])

    f = pl.kernel(
        body,
        out_type=jax.ShapeDtypeStruct((2, _N, 16), _F32),
        mesh=_sc_mesh(),
        compiler_params=pltpu.CompilerParams(use_tc_tiling_on_sc=False),
        scratch_types=[
            pltpu.VMEM((_SRPT, _SRPD), jnp.int32),
            pltpu.VMEM((2 * _GCH, 16), _F32),
            pltpu.VMEM_SHARED((_N, 16), _F32),
            pltpu.SemaphoreType.DMA,
            pltpu.SemaphoreType.DMA,
        ],
    )
    return f(pe, dst2, zeros)


def kernel(current_state, edge_index, efeatures, params):
    src = edge_index[0].astype(jnp.int32)
    dst = edge_index[1].astype(jnp.int32)
    dst2 = dst.reshape(_SROW, _SRPD)
    zeros = jnp.zeros((_N, 16), _F32)
    n8 = _N // 8

    ef_p = efeatures.reshape(_E // 8, 128)
    pe_red_p, pe_rec_p = _edge_encoder(ef_p, params["enc_e_red"],
                                       params["enc_e_rec"])
    pn = _node_encoder(current_state, params["enc_n_red"])
    pn_p = pn.reshape(n8, 128)

    def mp_pass(pn_p, pe_p, e_params, n_params):
        for i in range(3):
            gs, gd = _sc_gather(pn_p.reshape(_N, 16), src, dst)
            pe_p = _edge_proc(pe_p, gs.reshape(_E // 8, 128),
                              gd.reshape(_E // 8, 128), e_params[i])
            agg2 = _sc_scatter(pe_p.reshape(_E, 16), dst2, zeros)
            pn_p = _node_proc(pn_p, agg2.reshape(2, n8, 128), n_params[i])
        return pn_p

    pn_p = mp_pass(pn_p, pe_red_p, params["proc_e_red"], params["proc_n_red"])
    pn_p = _mid_decoder_encoder(pn_p, params["dec_n_red"], params["enc_n_rec"])
    pn_p = mp_pass(pn_p, pe_rec_p, params["proc_e_rec"], params["proc_n_rec"])
    out_p = _final_decoder(pn_p, params["dec_n_rec"])
    return out_p.reshape(_N, 2)
